# baseline (speedup 1.0000x reference)
import jax
import jax.numpy as jnp
from jax.experimental import pallas as pl
from jax.experimental.pallas import tpu as pltpu

_SIZE = 26
_D = 2048
_LEVELS = 100
_LP = 104
_K = _SIZE * _LP
_BATCH = 1024
_BB = 256
_NB = _BATCH // _BB


def _body(x_ref, pos_ref, lev_ref, out_ref, w_ref, e_ref, lmod_ref, oh_ref):
    i = pl.program_id(0)

    def _make_oh(blk, slot):
        xb = x_ref[pl.ds(blk * _BB, _BB), :]
        idx = jnp.clip(jnp.round(xb * (_LEVELS - 1)).astype(jnp.int32),
                       0, _LEVELS - 1)
        idx32 = jnp.pad(idx.astype(jnp.bfloat16), ((0, 0), (0, 32 - _SIZE)))
        expand = jnp.dot(idx32, e_ref[...],
                         preferred_element_type=jnp.float32)
        oh_ref[slot] = (expand == lmod_ref[...]).astype(jnp.bfloat16)

    @pl.when(i == 0)
    def _build_tables():
        lev = lev_ref[...]
        rows = jax.lax.broadcasted_iota(jnp.int32, (_LP, _D), 0)
        lev = jnp.where(rows < _LEVELS, lev, 0.0)
        for s in range(_SIZE):
            p = pos_ref[s:s + 1, :]
            w_ref[s * _LP:(s + 1) * _LP, :] = (lev * p).astype(jnp.bfloat16)
        js = jax.lax.broadcasted_iota(jnp.int32, (32, _K), 1) // _LP
        ss = jax.lax.broadcasted_iota(jnp.int32, (32, _K), 0)
        e_ref[...] = jnp.where(js == ss, 1.0, 0.0).astype(jnp.bfloat16)
        cols = jax.lax.broadcasted_iota(jnp.int32, (_BB, _K), 1)
        lmod_ref[...] = (cols % _LP).astype(jnp.float32)
        _make_oh(0, 0)

    @pl.when(i < _NB - 1)
    def _next_oh():
        _make_oh(i + 1, (i + 1) % 2)

    oh = oh_ref[i % 2]
    acc = jnp.dot(oh, w_ref[...], preferred_element_type=jnp.float32)
    out_ref[...] = jnp.where(acc > 0, 1.0, -1.0).astype(jnp.float32)


def kernel(x, position_weight, level_weight):
    return pl.pallas_call(
        _body,
        grid=(_NB,),
        in_specs=[
            pl.BlockSpec((_BATCH, _SIZE), lambda i: (0, 0)),
            pl.BlockSpec((32, _D), lambda i: (0, 0)),
            pl.BlockSpec((_LP, _D), lambda i: (0, 0)),
        ],
        out_specs=pl.BlockSpec((_BB, _D), lambda i: (i, 0)),
        out_shape=jax.ShapeDtypeStruct((_BATCH, _D), jnp.float32),
        scratch_shapes=[pltpu.VMEM((_K, _D), jnp.bfloat16),
                        pltpu.VMEM((32, _K), jnp.bfloat16),
                        pltpu.VMEM((_BB, _K), jnp.float32),
                        pltpu.VMEM((2, _BB, _K), jnp.bfloat16)],
    )(x, position_weight, level_weight)
